# baseline (speedup 1.0000x reference)
import functools

import jax
import jax.numpy as jnp
from jax.experimental import pallas as pl
from jax.experimental.pallas import tpu as pltpu


def _silu(x):
    return x * jax.nn.sigmoid(x)


def _moe_body(x_ref, gate_ref, sgw_ref, w13_ref, w2_ref, wg_ref, wu_ref,
              wd_ref, out_ref, comb_ref, sg_ref, *, TM, E, I):
    e = pl.program_id(0)
    mt = pl.program_id(1)

    xb = x_ref[pl.ds(mt * TM, TM), :]
    xb16 = xb.astype(jnp.bfloat16)

    @pl.when(e == 0)
    def _router():
        logits = jax.lax.dot_general(
            xb16, gate_ref[...].astype(jnp.bfloat16), (((1,), (1,)), ((), ())),
            preferred_element_type=jnp.float32)
        w = jax.nn.softmax(logits, axis=-1)
        iota = jax.lax.broadcasted_iota(jnp.int32, w.shape, 1)
        m1 = jnp.max(w, axis=-1, keepdims=True)
        i1 = jnp.min(jnp.where(w == m1, iota, E), axis=-1, keepdims=True)
        wm = jnp.where(iota == i1, -1.0, w)
        m2 = jnp.max(wm, axis=-1, keepdims=True)
        i2 = jnp.min(jnp.where(wm == m2, iota, E), axis=-1, keepdims=True)
        comb_ref[pl.ds(mt * TM, TM), :] = jnp.where(
            (iota == i1) | (iota == i2), w, 0.0)
        sgw16 = sgw_ref[...].astype(jnp.bfloat16).astype(jnp.float32)
        sgl = jnp.sum(xb16.astype(jnp.float32) * sgw16, axis=-1,
                      keepdims=True)
        sg_ref[pl.ds(mt * TM, TM), :] = jnp.broadcast_to(
            jax.nn.sigmoid(sgl), (TM, 8))

    combine = comb_ref[pl.ds(mt * TM, TM), :]
    iota = jax.lax.broadcasted_iota(jnp.int32, combine.shape, 1)
    ce = jnp.sum(jnp.where(iota == e, combine, 0.0), axis=-1,
                 keepdims=True)

    w13 = w13_ref[0].astype(jnp.bfloat16)
    h = jax.lax.dot_general(xb16, w13, (((1,), (1,)), ((), ())),
                            preferred_element_type=jnp.float32)
    act = (_silu(h[:, :I]) * h[:, I:]).astype(jnp.bfloat16)
    w2 = w2_ref[0].astype(jnp.bfloat16)
    eo = jax.lax.dot_general(act, w2, (((1,), (1,)), ((), ())),
                             preferred_element_type=jnp.float32)
    acc = ce * eo

    wg = wg_ref[...].astype(jnp.bfloat16)
    wu = wu_ref[...].astype(jnp.bfloat16)
    gs = jax.lax.dot_general(xb16, wg, (((1,), (1,)), ((), ())),
                             preferred_element_type=jnp.float32)
    us = jax.lax.dot_general(xb16, wu, (((1,), (1,)), ((), ())),
                             preferred_element_type=jnp.float32)
    sa = (_silu(gs) * us).astype(jnp.bfloat16)
    wd = wd_ref[...].astype(jnp.bfloat16)
    so = jax.lax.dot_general(sa, wd, (((1,), (1,)), ((), ())),
                             preferred_element_type=jnp.float32)
    sgv = sg_ref[pl.ds(mt * TM, TM), 0:1]
    acc = acc + so * sgv

    @pl.when(e == 0)
    def _init():
        out_ref[pl.ds(mt * TM, TM), :] = acc

    @pl.when(e != 0)
    def _accum():
        out_ref[pl.ds(mt * TM, TM), :] = out_ref[pl.ds(mt * TM, TM), :] + acc


def kernel(hidden_states, w13_stacked, w2_stacked, gate_w,
           shared_expert_gate_w, shared_gate_up_w, shared_down_w):
    orig_shape = hidden_states.shape
    H = orig_shape[-1]
    x = hidden_states.reshape(-1, H)
    M = x.shape[0]
    E, twoI, _ = w13_stacked.shape
    I = twoI // 2
    S = shared_down_w.shape[1]
    TS = S // E
    TM = min(256, M)
    MT = M // TM

    grid = (E, MT)
    out = pl.pallas_call(
        functools.partial(_moe_body, TM=TM, E=E, I=I),
        grid=grid,
        in_specs=[
            pl.BlockSpec((M, H), lambda e, mt: (0, 0)),
            pl.BlockSpec((E, H), lambda e, mt: (0, 0)),
            pl.BlockSpec((1, H), lambda e, mt: (0, 0)),
            pl.BlockSpec((1, twoI, H), lambda e, mt: (e, 0, 0)),
            pl.BlockSpec((1, H, I), lambda e, mt: (e, 0, 0)),
            pl.BlockSpec((TS, H), lambda e, mt: (e, 0)),
            pl.BlockSpec((TS, H), lambda e, mt: (e + E, 0)),
            pl.BlockSpec((H, TS), lambda e, mt: (0, e)),
        ],
        out_specs=pl.BlockSpec((M, H), lambda e, mt: (0, 0)),
        out_shape=jax.ShapeDtypeStruct((M, H), jnp.float32),
        scratch_shapes=[
            pltpu.VMEM((M, 8), jnp.float32),
            pltpu.VMEM((M, 8), jnp.float32),
        ],
        compiler_params=pltpu.CompilerParams(
            dimension_semantics=("arbitrary", "arbitrary")),
    )(x, gate_w, shared_expert_gate_w, w13_stacked, w2_stacked,
      shared_gate_up_w, shared_gate_up_w, shared_down_w)
    return out.reshape(orig_shape)


# speedup vs baseline: 1.3016x; 1.3016x over previous
import functools

import jax
import jax.numpy as jnp
from jax.experimental import pallas as pl
from jax.experimental.pallas import tpu as pltpu


def _silu(x):
    return x * jax.nn.sigmoid(x)


def _moe_body(x_ref, gate_ref, sgw_ref, w13_ref, w2_ref, wg_ref, wu_ref,
              wd_ref, out_ref, comb_ref, sg_ref, xb16_ref, w13b_ref, w2b_ref,
              wgb_ref, wub_ref, wdb_ref, *, TM, E, I, TS):
    e = pl.program_id(0)
    mt = pl.program_id(1)

    @pl.when(e == 0)
    def _prep_x():
        xb = x_ref[pl.ds(mt * TM, TM), :]
        xb16_ref[pl.ds(mt * TM, TM), :] = xb.astype(jnp.bfloat16)

    @pl.when(mt == 0)
    def _prep_w():
        w13b_ref[...] = w13_ref[0].astype(jnp.bfloat16)
        w2b_ref[...] = w2_ref[0].astype(jnp.bfloat16)
        wgb_ref[...] = wg_ref[...].astype(jnp.bfloat16)
        wub_ref[...] = wu_ref[...].astype(jnp.bfloat16)
        wdb_ref[...] = wd_ref[...].astype(jnp.bfloat16)

    xb16 = xb16_ref[pl.ds(mt * TM, TM), :]

    @pl.when(e == 0)
    def _router():
        logits = jax.lax.dot_general(
            xb16, gate_ref[...].astype(jnp.bfloat16), (((1,), (1,)), ((), ())),
            preferred_element_type=jnp.float32)
        w = jax.nn.softmax(logits, axis=-1)
        iota = jax.lax.broadcasted_iota(jnp.int32, w.shape, 1)
        m1 = jnp.max(w, axis=-1, keepdims=True)
        i1 = jnp.min(jnp.where(w == m1, iota, E), axis=-1, keepdims=True)
        wm = jnp.where(iota == i1, -1.0, w)
        m2 = jnp.max(wm, axis=-1, keepdims=True)
        i2 = jnp.min(jnp.where(wm == m2, iota, E), axis=-1, keepdims=True)
        comb_ref[pl.ds(mt * TM, TM), :] = jnp.where(
            (iota == i1) | (iota == i2), w, 0.0)
        sgw16 = sgw_ref[...].astype(jnp.bfloat16).astype(jnp.float32)
        sgl = jnp.sum(xb16.astype(jnp.float32) * sgw16, axis=-1,
                      keepdims=True)
        sg_ref[pl.ds(mt * TM, TM), :] = jnp.broadcast_to(
            jax.nn.sigmoid(sgl), (TM, 8))

    combine = comb_ref[pl.ds(mt * TM, TM), :]
    iota = jax.lax.broadcasted_iota(jnp.int32, combine.shape, 1)
    ce = jnp.sum(jnp.where(iota == e, combine, 0.0), axis=-1,
                 keepdims=True)

    h = jax.lax.dot_general(xb16, w13b_ref[...], (((1,), (1,)), ((), ())),
                            preferred_element_type=jnp.float32)
    act = (_silu(h[:, :I]) * h[:, I:]).astype(jnp.bfloat16)
    eo = jax.lax.dot_general(act, w2b_ref[...], (((1,), (1,)), ((), ())),
                             preferred_element_type=jnp.float32)
    acc = ce * eo

    gs = jax.lax.dot_general(xb16, wgb_ref[...], (((1,), (1,)), ((), ())),
                             preferred_element_type=jnp.float32)
    us = jax.lax.dot_general(xb16, wub_ref[...], (((1,), (1,)), ((), ())),
                             preferred_element_type=jnp.float32)
    sa = (_silu(gs) * us).astype(jnp.bfloat16)
    so = jax.lax.dot_general(sa, wdb_ref[...], (((1,), (1,)), ((), ())),
                             preferred_element_type=jnp.float32)
    sgv = sg_ref[pl.ds(mt * TM, TM), 0:1]
    acc = acc + so * sgv

    @pl.when(e == 0)
    def _init():
        out_ref[pl.ds(mt * TM, TM), :] = acc

    @pl.when(e != 0)
    def _accum():
        out_ref[pl.ds(mt * TM, TM), :] = out_ref[pl.ds(mt * TM, TM), :] + acc


def kernel(hidden_states, w13_stacked, w2_stacked, gate_w,
           shared_expert_gate_w, shared_gate_up_w, shared_down_w):
    orig_shape = hidden_states.shape
    H = orig_shape[-1]
    x = hidden_states.reshape(-1, H)
    M = x.shape[0]
    E, twoI, _ = w13_stacked.shape
    I = twoI // 2
    S = shared_down_w.shape[1]
    TS = S // E
    TM = min(1024, M)
    MT = M // TM

    grid = (E, MT)
    out = pl.pallas_call(
        functools.partial(_moe_body, TM=TM, E=E, I=I, TS=TS),
        grid=grid,
        in_specs=[
            pl.BlockSpec((M, H), lambda e, mt: (0, 0)),
            pl.BlockSpec((E, H), lambda e, mt: (0, 0)),
            pl.BlockSpec((1, H), lambda e, mt: (0, 0)),
            pl.BlockSpec((1, twoI, H), lambda e, mt: (e, 0, 0)),
            pl.BlockSpec((1, H, I), lambda e, mt: (e, 0, 0)),
            pl.BlockSpec((TS, H), lambda e, mt: (e, 0)),
            pl.BlockSpec((TS, H), lambda e, mt: (e + E, 0)),
            pl.BlockSpec((H, TS), lambda e, mt: (0, e)),
        ],
        out_specs=pl.BlockSpec((M, H), lambda e, mt: (0, 0)),
        out_shape=jax.ShapeDtypeStruct((M, H), jnp.float32),
        scratch_shapes=[
            pltpu.VMEM((M, 8), jnp.float32),
            pltpu.VMEM((M, 8), jnp.float32),
            pltpu.VMEM((M, H), jnp.bfloat16),
            pltpu.VMEM((twoI, H), jnp.bfloat16),
            pltpu.VMEM((H, I), jnp.bfloat16),
            pltpu.VMEM((TS, H), jnp.bfloat16),
            pltpu.VMEM((TS, H), jnp.bfloat16),
            pltpu.VMEM((H, TS), jnp.bfloat16),
        ],
        compiler_params=pltpu.CompilerParams(
            dimension_semantics=("arbitrary", "arbitrary")),
    )(x, gate_w, shared_expert_gate_w, w13_stacked, w2_stacked,
      shared_gate_up_w, shared_gate_up_w, shared_down_w)
    return out.reshape(orig_shape)
